# Initial kernel scaffold; baseline (speedup 1.0000x reference)
import jax
import jax.numpy as jnp
from jax.experimental import pallas as pl


def kernel(x_customer, x_product, edge_index_buys, W1b_src, W1b_dst, a1b_src, a1b_dst, b1b, W1r_src, W1r_dst, a1r_src, a1r_dst, b1r, W2b_src, W2b_dst, a2b_src, a2b_dst, b2b, W2r_src, W2r_dst, a2r_src, a2r_dst, b2r):
    raise NotImplementedError("write your pallas kernel here")


# speedup vs baseline: 1.1667x; 1.1667x over previous
import jax
import jax.numpy as jnp
from jax.experimental import pallas as pl

NC = 50000
NP = 50000
E = 300000
HID = 64
HEADS1 = 2
OUT = 64


def _gat(x_src, x_dst, edge_index, W_src, W_dst, att_src, att_dst, bias, heads, ch, num_dst):
    hs = (x_src @ W_src).reshape(-1, heads, ch)
    hd = (x_dst @ W_dst).reshape(-1, heads, ch)
    a_s = (hs * att_src).sum(-1)
    a_d = (hd * att_dst).sum(-1)
    src = edge_index[0]
    dst = edge_index[1]
    alpha = jax.nn.leaky_relu(a_s[src] + a_d[dst], negative_slope=0.2)
    w = jnp.exp(alpha)
    den = jax.ops.segment_sum(w, dst, num_segments=num_dst)
    msg = hs[src] * w[:, :, None]
    num = jax.ops.segment_sum(msg, dst, num_segments=num_dst)
    den = den[:, :, None]
    out = jnp.where(den > 0, num / jnp.where(den > 0, den, 1.0), 0.0)
    return out.reshape(num_dst, heads * ch) + bias


def _bias_relu_kernel(x_ref, b_ref, o_ref, *, relu):
    v = x_ref[...] + b_ref[...]
    o_ref[...] = jnp.maximum(v, 0.0) if relu else v


def _bias(x, b, relu):
    n, f = x.shape
    blk = 400
    return pl.pallas_call(
        lambda x_ref, b_ref, o_ref: _bias_relu_kernel(x_ref, b_ref, o_ref, relu=relu),
        grid=(n // blk,),
        in_specs=[pl.BlockSpec((blk, f), lambda i: (i, 0)),
                  pl.BlockSpec((1, f), lambda i: (0, 0))],
        out_specs=pl.BlockSpec((blk, f), lambda i: (i, 0)),
        out_shape=jax.ShapeDtypeStruct((n, f), x.dtype),
    )(x, b.reshape(1, f))


def kernel(x_customer, x_product, edge_index_buys,
           W1b_src, W1b_dst, a1b_src, a1b_dst, b1b,
           W1r_src, W1r_dst, a1r_src, a1r_dst, b1r,
           W2b_src, W2b_dst, a2b_src, a2b_dst, b2b,
           W2r_src, W2r_dst, a2r_src, a2r_dst, b2r):
    ei_rev = jnp.flip(edge_index_buys, axis=0)
    p1 = _gat(x_customer, x_product, edge_index_buys, W1b_src, W1b_dst, a1b_src, a1b_dst, jnp.zeros_like(b1b), HEADS1, HID, NP)
    c1 = _gat(x_product, x_customer, ei_rev, W1r_src, W1r_dst, a1r_src, a1r_dst, jnp.zeros_like(b1r), HEADS1, HID, NC)
    p1 = _bias(p1, b1b, relu=True)
    c1 = _bias(c1, b1r, relu=True)
    out_p = _gat(c1, p1, edge_index_buys, W2b_src, W2b_dst, a2b_src, a2b_dst, jnp.zeros_like(b2b), 1, OUT, NP)
    out_c = _gat(p1, c1, ei_rev, W2r_src, W2r_dst, a2r_src, a2r_dst, jnp.zeros_like(b2r), 1, OUT, NC)
    out_p = _bias(out_p, b2b, relu=False)
    out_c = _bias(out_c, b2r, relu=False)
    return (out_c, out_p)


# speedup vs baseline: 13.3072x; 11.4062x over previous
import functools

import jax
import jax.numpy as jnp
from jax import lax
from jax.experimental import pallas as pl
from jax.experimental.pallas import tpu as pltpu
from jax.experimental.pallas import tpu_sc as plsc

N = 50000
E = 300000
D_IN = 128
HID = 64
H1 = 2
OUT = 64
F1 = H1 * HID
RB = 400
NBLK = N // RB

EB = 128
NFULL = E // EB
REM = E - NFULL * EB
TAIL_B = NFULL * EB

f32 = jnp.float32
i32 = jnp.int32


def _heads_dot(hs, att, heads, ch):
    cols = [jnp.sum(hs[:, h * ch:(h + 1) * ch] * att[h][None, :], axis=1,
                    keepdims=True) for h in range(heads)]
    return jnp.concatenate(cols, axis=1)


def _fold(W, att, heads, ch):
    cols = [jnp.sum(W[:, h * ch:(h + 1) * ch] * att[h][None, :], axis=1,
                    keepdims=True) for h in range(heads)]
    return jnp.concatenate(cols, axis=1)


def _att_pack(a_s, a_d, heads):
    z = jnp.zeros((a_s.shape[0], 1), f32)
    pads = [z] * (4 - heads)
    pade = [z] * (12 - heads)
    return jnp.concatenate([a_s] + pads + [a_d] + pade, axis=1)


def _tc1_body(xc_ref, xp_ref, wbs_ref, abs_ref, wrd_ref, ard_ref,
              wrs_ref, ars_ref, wbd_ref, abd_ref,
              hsb_ref, attc_ref, hsr_ref, attp_ref):
    xc = xc_ref[...]
    xp = xp_ref[...]
    hsb = jnp.dot(xc, wbs_ref[...], preferred_element_type=f32)
    for c in range(4):
        hsb_ref[c] = hsb[:, 32 * c:32 * c + 32]
    a_s_b = _heads_dot(hsb, abs_ref[...], H1, HID)
    a_d_r = jnp.dot(xc, _fold(wrd_ref[...], ard_ref[...], H1, HID),
                    preferred_element_type=f32)
    attc_ref[...] = _att_pack(a_s_b, a_d_r, H1)
    hsr = jnp.dot(xp, wrs_ref[...], preferred_element_type=f32)
    for c in range(4):
        hsr_ref[c] = hsr[:, 32 * c:32 * c + 32]
    a_s_r = _heads_dot(hsr, ars_ref[...], H1, HID)
    a_d_b = jnp.dot(xp, _fold(wbd_ref[...], abd_ref[...], H1, HID),
                    preferred_element_type=f32)
    attp_ref[...] = _att_pack(a_s_r, a_d_b, H1)


def _tc1(xc, xp, W1b_src, a1b_src, W1r_dst, a1r_dst,
         W1r_src, a1r_src, W1b_dst, a1b_dst):
    full2 = lambda shape: pl.BlockSpec(shape, lambda i: tuple(0 for _ in shape))
    row = pl.BlockSpec((RB, D_IN), lambda i: (i, 0))
    return pl.pallas_call(
        _tc1_body,
        grid=(NBLK,),
        in_specs=[row, row, full2((D_IN, F1)), full2((H1, HID)),
                  full2((D_IN, F1)), full2((H1, HID)),
                  full2((D_IN, F1)), full2((H1, HID)),
                  full2((D_IN, F1)), full2((H1, HID))],
        out_specs=[pl.BlockSpec((4, RB, 32), lambda i: (0, i, 0)),
                   pl.BlockSpec((RB, 16), lambda i: (i, 0)),
                   pl.BlockSpec((4, RB, 32), lambda i: (0, i, 0)),
                   pl.BlockSpec((RB, 16), lambda i: (i, 0))],
        out_shape=[jax.ShapeDtypeStruct((4, N, 32), f32),
                   jax.ShapeDtypeStruct((N, 16), f32),
                   jax.ShapeDtypeStruct((4, N, 32), f32),
                   jax.ShapeDtypeStruct((N, 16), f32)],
    )(xc, xp, W1b_src, a1b_src, W1r_dst, a1r_dst,
      W1r_src, a1r_src, W1b_dst, a1b_dst)


def _normalize(acc_ref, den_ref, bias_ref, nch, relu):
    den = den_ref[0] + den_ref[1]
    cols = []
    for c in range(nch):
        h = c // 2 if nch == 4 else 0
        d = den[:, h:h + 1]
        d_safe = jnp.where(d > 0, d, 1.0)
        cols.append(jnp.where(d > 0, acc_ref[c] / d_safe, 0.0))
    out = jnp.concatenate(cols, axis=1) + bias_ref[...]
    return jnp.maximum(out, 0.0) if relu else out


def _mid_body(accb_ref, denb_ref, b1b_ref, accr_ref, denr_ref, b1r_ref,
              w2bs_ref, a2bs_ref, w2rd_ref, a2rd_ref,
              w2rs_ref, a2rs_ref, w2bd_ref, a2bd_ref,
              hs2b_ref, att2c_ref, hs2r_ref, att2p_ref):
    p1 = _normalize(accb_ref, denb_ref, b1b_ref, 4, True)
    c1 = _normalize(accr_ref, denr_ref, b1r_ref, 4, True)
    hs2b = jnp.dot(c1, w2bs_ref[...], preferred_element_type=f32)
    for c in range(2):
        hs2b_ref[c] = hs2b[:, 32 * c:32 * c + 32]
    a_s_b = _heads_dot(hs2b, a2bs_ref[...], 1, OUT)
    a_d_r = jnp.dot(c1, _fold(w2rd_ref[...], a2rd_ref[...], 1, OUT),
                    preferred_element_type=f32)
    att2c_ref[...] = _att_pack(a_s_b, a_d_r, 1)
    hs2r = jnp.dot(p1, w2rs_ref[...], preferred_element_type=f32)
    for c in range(2):
        hs2r_ref[c] = hs2r[:, 32 * c:32 * c + 32]
    a_s_r = _heads_dot(hs2r, a2rs_ref[...], 1, OUT)
    a_d_b = jnp.dot(p1, _fold(w2bd_ref[...], a2bd_ref[...], 1, OUT),
                    preferred_element_type=f32)
    att2p_ref[...] = _att_pack(a_s_r, a_d_b, 1)


def _mid(accb, denb, b1b, accr, denr, b1r,
         W2b_src, a2b_src, W2r_dst, a2r_dst, W2r_src, a2r_src,
         W2b_dst, a2b_dst):
    full2 = lambda shape: pl.BlockSpec(shape, lambda i: tuple(0 for _ in shape))
    acc_s = pl.BlockSpec((4, RB, 32), lambda i: (0, i, 0))
    den_s = pl.BlockSpec((2, RB, 8), lambda i: (0, i, 0))
    return pl.pallas_call(
        _mid_body,
        grid=(NBLK,),
        in_specs=[acc_s, den_s, full2((1, F1)), acc_s, den_s, full2((1, F1)),
                  full2((F1, OUT)), full2((1, OUT)),
                  full2((F1, OUT)), full2((1, OUT)),
                  full2((F1, OUT)), full2((1, OUT)),
                  full2((F1, OUT)), full2((1, OUT))],
        out_specs=[pl.BlockSpec((2, RB, 32), lambda i: (0, i, 0)),
                   pl.BlockSpec((RB, 16), lambda i: (i, 0)),
                   pl.BlockSpec((2, RB, 32), lambda i: (0, i, 0)),
                   pl.BlockSpec((RB, 16), lambda i: (i, 0))],
        out_shape=[jax.ShapeDtypeStruct((2, N, 32), f32),
                   jax.ShapeDtypeStruct((N, 16), f32),
                   jax.ShapeDtypeStruct((2, N, 32), f32),
                   jax.ShapeDtypeStruct((N, 16), f32)],
    )(accb, denb, b1b, accr, denr, b1r,
      W2b_src, a2b_src, W2r_dst, a2r_dst, W2r_src, a2r_src,
      W2b_dst, a2b_dst)


def _final_body(acc2b_ref, den2b_ref, b2b_ref, acc2r_ref, den2r_ref, b2r_ref,
                outp_ref, outc_ref):
    outp_ref[...] = _normalize(acc2b_ref, den2b_ref, b2b_ref, 2, False)
    outc_ref[...] = _normalize(acc2r_ref, den2r_ref, b2r_ref, 2, False)


def _final(acc2b, den2b, b2b, acc2r, den2r, b2r):
    full2 = lambda shape: pl.BlockSpec(shape, lambda i: tuple(0 for _ in shape))
    acc_s = pl.BlockSpec((2, RB, 32), lambda i: (0, i, 0))
    den_s = pl.BlockSpec((2, RB, 8), lambda i: (0, i, 0))
    return pl.pallas_call(
        _final_body,
        grid=(NBLK,),
        in_specs=[acc_s, den_s, full2((1, OUT)), acc_s, den_s, full2((1, OUT))],
        out_specs=[pl.BlockSpec((RB, OUT), lambda i: (i, 0)),
                   pl.BlockSpec((RB, OUT), lambda i: (i, 0))],
        out_shape=[jax.ShapeDtypeStruct((N, OUT), f32),
                   jax.ShapeDtypeStruct((N, OUT), f32)],
    )(acc2b, den2b, b2b, acc2r, den2r, b2r)


_MESH = plsc.VectorSubcoreMesh(core_axis_name="c", subcore_axis_name="s")
_IOTA = lambda: lax.iota(i32, 16)

DEN_W = N * 8
DEN_TILE = DEN_W // 16
ACC_ROWS = N // 16


def _p1_batch(b, B, att_s, att_d, src, dst, w_hbm, heads,
              srcb, dstb, rows_s, rows_d, wbuf, dix0, dix1):
    pltpu.sync_copy(src.at[pl.ds(b * EB, B)], srcb)
    pltpu.sync_copy(dst.at[pl.ds(b * EB, B)], dstb)
    pltpu.sync_copy(att_s.at[srcb], rows_s)
    pltpu.sync_copy(att_d.at[dstb], rows_d)
    dixs = (dix0, dix1)
    for h in range(heads):
        for g in range(B // 16):
            rid = _IOTA() + g * 16
            sv = plsc.load_gather(rows_s, [rid, jnp.full((16,), h, i32)])
            dv = plsc.load_gather(rows_d, [rid, jnp.full((16,), 4 + h, i32)])
            al = sv + dv
            al = jnp.maximum(al, al * 0.2)
            wbuf[pl.ds(h * EB + g * 16, 16)] = jnp.exp(al)
            didx = dstb[pl.ds(g * 16, 16)]
            dixs[h][pl.ds(g * 16, 16)] = didx * 8 + h
    for h in range(heads):
        pltpu.sync_copy(wbuf.at[pl.ds(h * EB, B)],
                        w_hbm.at[pl.ds(h * E + b * EB, B)])


def _make_p1(heads):
    scratch = [
        pltpu.VMEM((EB,), i32),
        pltpu.VMEM((EB,), i32),
        pltpu.VMEM((REM,), i32),
        pltpu.VMEM((REM,), i32),
        pltpu.VMEM((EB, 16), f32),
        pltpu.VMEM((EB, 16), f32),
        pltpu.VMEM((H1 * EB,), f32),
        pltpu.VMEM((EB,), i32),
        pltpu.VMEM((EB,), i32),
        pltpu.VMEM((REM,), i32),
        pltpu.VMEM((REM,), i32),
        pltpu.VMEM((2000,), f32),
        pltpu.VMEM_SHARED((DEN_W,), f32),
    ]

    @functools.partial(
        pl.kernel,
        out_type=[jax.ShapeDtypeStruct((heads * E,), f32),
                  jax.ShapeDtypeStruct((2 * DEN_W,), f32)],
        mesh=_MESH,
        scratch_types=scratch,
        compiler_params=pltpu.CompilerParams(
            needs_layout_passes=False, use_tc_tiling_on_sc=False),
    )
    def p1(att_s, att_d, src, dst, w_hbm, den_hbm,
           srcb, dstb, srcb_r, dstb_r, rows_s, rows_d, wbuf,
           dix0, dix1, dix0_r, dix1_r, zbuf, den_sp):
        cid = lax.axis_index("c")
        tid = lax.axis_index("s")
        wid = tid * 2 + cid
        for k in range(125):
            zbuf[pl.ds(k * 16, 16)] = jnp.zeros((16,), f32)
        base = tid * DEN_TILE
        for j in range(12):
            pltpu.sync_copy(zbuf, den_sp.at[pl.ds(base + j * 2000, 2000)])
        pltpu.sync_copy(zbuf.at[pl.ds(0, 1000)],
                        den_sp.at[pl.ds(base + 24000, 1000)])
        plsc.subcore_barrier()

        def body(i, carry):
            b = wid + i * 32

            @pl.when(b < NFULL)
            def _():
                _p1_batch(b, EB, att_s, att_d, src, dst, w_hbm, heads,
                          srcb, dstb, rows_s, rows_d, wbuf, dix0, dix1)
                pltpu.sync_copy(wbuf.at[pl.ds(0, EB)], den_sp.at[dix0],
                                add=True)
                if heads == 2:
                    pltpu.sync_copy(wbuf.at[pl.ds(EB, EB)], den_sp.at[dix1],
                                    add=True)
            return carry

        lax.fori_loop(0, 74, body, 0)

        @pl.when(wid == 7)
        def _():
            _p1_batch(NFULL, REM, att_s, att_d, src, dst, w_hbm, heads,
                      srcb_r, dstb_r, rows_s.at[pl.ds(0, REM)],
                      rows_d.at[pl.ds(0, REM)], wbuf, dix0_r, dix1_r)
            pltpu.sync_copy(wbuf.at[pl.ds(0, REM)], den_sp.at[dix0_r],
                            add=True)
            if heads == 2:
                pltpu.sync_copy(wbuf.at[pl.ds(EB, REM)], den_sp.at[dix1_r],
                                add=True)

        plsc.subcore_barrier()
        pltpu.sync_copy(den_sp.at[pl.ds(base, DEN_TILE)],
                        den_hbm.at[pl.ds(cid * DEN_W + base, DEN_TILE)])

    return p1


def _p2_batch(b, B, h, base, hs_hbm, w_hbm, src, dst,
              srcb, dstb, idxg, wb, rows, acc_sp):
    pltpu.sync_copy(src.at[pl.ds(b * EB, B)], srcb)
    pltpu.sync_copy(dst.at[pl.ds(b * EB, B)], dstb)
    pltpu.sync_copy(w_hbm.at[pl.ds(h * E + b * EB, B)], wb)
    for g in range(B // 16):
        sv = srcb[pl.ds(g * 16, 16)]
        idxg[pl.ds(g * 16, 16)] = sv + base
    pltpu.sync_copy(hs_hbm.at[idxg], rows)
    for g in range(B // 16):
        wv = wb[pl.ds(g * 16, 16)]
        rid = _IOTA() + g * 16
        for j in range(32):
            cj = jnp.full((16,), j, i32)
            v = plsc.load_gather(rows, [rid, cj])
            plsc.store_scatter(rows, [rid, cj], v * wv)
    pltpu.sync_copy(rows, acc_sp.at[dstb], add=True)


def _make_p2(nch):
    rounds = nch // 2
    scratch = [
        pltpu.VMEM((EB,), i32),
        pltpu.VMEM((EB,), i32),
        pltpu.VMEM((EB,), i32),
        pltpu.VMEM((REM,), i32),
        pltpu.VMEM((REM,), i32),
        pltpu.VMEM((REM,), i32),
        pltpu.VMEM((EB,), f32),
        pltpu.VMEM((EB, 32), f32),
        pltpu.VMEM((125, 32), f32),
        pltpu.VMEM_SHARED((N, 32), f32),
    ]

    @functools.partial(
        pl.kernel,
        out_type=jax.ShapeDtypeStruct((nch * N, 32), f32),
        mesh=_MESH,
        scratch_types=scratch,
        compiler_params=pltpu.CompilerParams(
            needs_layout_passes=False, use_tc_tiling_on_sc=False),
    )
    def p2(hs_hbm, w_hbm, src, dst, acc_hbm,
           srcb, dstb, idxg, srcb_r, dstb_r, idxg_r, wb, rows, zrows, acc_sp):
        cid = lax.axis_index("c")
        tid = lax.axis_index("s")
        for k in range(125):
            zrows[k, pl.ds(0, 16)] = jnp.zeros((16,), f32)
            zrows[k, pl.ds(16, 16)] = jnp.zeros((16,), f32)
        row0 = tid * ACC_ROWS
        for r in range(rounds):
            chunk = 2 * r + cid
            base = chunk * N
            for j in range(25):
                pltpu.sync_copy(zrows, acc_sp.at[pl.ds(row0 + j * 125, 125)])
            plsc.subcore_barrier()

            def body(i, carry):
                b = tid + i * 16

                @pl.when(b < NFULL)
                def _():
                    _p2_batch(b, EB, r, base, hs_hbm, w_hbm, src, dst,
                              srcb, dstb, idxg, wb, rows, acc_sp)
                return carry

            lax.fori_loop(0, 147, body, 0)

            @pl.when(tid == 7)
            def _():
                _p2_batch(NFULL, REM, r, base, hs_hbm, w_hbm, src, dst,
                          srcb_r, dstb_r, idxg_r, wb.at[pl.ds(0, REM)],
                          rows.at[pl.ds(0, REM)], acc_sp)

            plsc.subcore_barrier()
            pltpu.sync_copy(acc_sp.at[pl.ds(row0, ACC_ROWS)],
                            acc_hbm.at[pl.ds(base + row0, ACC_ROWS)])
            plsc.subcore_barrier()

    return p2


_P1_L1 = _make_p1(2)
_P1_L2 = _make_p1(1)
_P2_L1 = _make_p2(4)
_P2_L2 = _make_p2(2)


def kernel(x_customer, x_product, edge_index_buys,
           W1b_src, W1b_dst, a1b_src, a1b_dst, b1b,
           W1r_src, W1r_dst, a1r_src, a1r_dst, b1r,
           W2b_src, W2b_dst, a2b_src, a2b_dst, b2b,
           W2r_src, W2r_dst, a2r_src, a2r_dst, b2r):
    srcv = edge_index_buys[0]
    dstv = edge_index_buys[1]

    hsb4, attc, hsr4, attp = _tc1(
        x_customer, x_product,
        W1b_src, a1b_src.reshape(H1, HID), W1r_dst, a1r_dst.reshape(H1, HID),
        W1r_src, a1r_src.reshape(H1, HID), W1b_dst, a1b_dst.reshape(H1, HID))

    w_b, den_b = _P1_L1(attc, attp, srcv, dstv)
    w_r, den_r = _P1_L1(attp, attc, dstv, srcv)
    accb = _P2_L1(hsb4.reshape(4 * N, 32), w_b, srcv, dstv)
    accr = _P2_L1(hsr4.reshape(4 * N, 32), w_r, dstv, srcv)

    hs2b2, att2c, hs2r2, att2p = _mid(
        accb.reshape(4, N, 32), den_b.reshape(2, N, 8), b1b.reshape(1, F1),
        accr.reshape(4, N, 32), den_r.reshape(2, N, 8), b1r.reshape(1, F1),
        W2b_src, a2b_src.reshape(1, OUT), W2r_dst, a2r_dst.reshape(1, OUT),
        W2r_src, a2r_src.reshape(1, OUT), W2b_dst, a2b_dst.reshape(1, OUT))

    w2b, den2b = _P1_L2(att2c, att2p, srcv, dstv)
    w2r, den2r = _P1_L2(att2p, att2c, dstv, srcv)
    acc2b = _P2_L2(hs2b2.reshape(2 * N, 32), w2b, srcv, dstv)
    acc2r = _P2_L2(hs2r2.reshape(2 * N, 32), w2r, dstv, srcv)

    out_p, out_c = _final(
        acc2b.reshape(2, N, 32), den2b.reshape(2, N, 8), b2b.reshape(1, OUT),
        acc2r.reshape(2, N, 32), den2r.reshape(2, N, 8), b2r.reshape(1, OUT))
    return (out_c, out_p)
